# baseline (speedup 1.0000x reference)
import functools

import jax
import jax.numpy as jnp
from jax import lax
from jax.experimental import pallas as pl
from jax.experimental.pallas import tpu as pltpu
from jax.experimental.pallas import tpu_sc as plsc

_NC = 2
_NS = 16
_NW = _NC * _NS
_B = 16384
_BPW = _B // _NW
_CW = 128
_KC = _BPW // _CW
_L = 16
_VPC = _CW // _L


def _make_sc_kernel():
    mesh = plsc.VectorSubcoreMesh(core_axis_name="c", subcore_axis_name="s")

    @functools.partial(
        pl.kernel,
        mesh=mesh,
        out_type=jax.ShapeDtypeStruct((2 * _NW * _L,), jnp.float32),
        scratch_types=[
            pltpu.VMEM((2, _KC, _CW), jnp.int32),
            pltpu.VMEM((3, _KC, _CW), jnp.float32),
            pltpu.VMEM((2, _L), jnp.float32),
            pltpu.SemaphoreType.DMA,
            pltpu.SemaphoreType.DMA,
        ],
    )
    def _k(su_hbm, si_hbm, r_hbm, bu_hbm, bi_hbm, out_hbm,
           idx_v, dat_v, st_v, sem_in, sem_g):
        cid = lax.axis_index("c")
        sid = lax.axis_index("s")
        wid = sid * _NC + cid

        ld_u = pltpu.async_copy(su_hbm.at[wid], idx_v.at[0], sem_in)
        ld_i = pltpu.async_copy(si_hbm.at[wid], idx_v.at[1], sem_in)
        ld_r = pltpu.async_copy(r_hbm.at[wid], dat_v.at[2], sem_in)
        ld_u.wait()
        ld_i.wait()
        copies = []
        for k in range(_KC):
            copies.append(pltpu.async_copy(
                bu_hbm.at[idx_v.at[0, k]], dat_v.at[0, k], sem_g))
            copies.append(pltpu.async_copy(
                bi_hbm.at[idx_v.at[1, k]], dat_v.at[1, k], sem_g))
        ld_r.wait()
        for c in copies:
            c.wait()

        acc1 = jnp.zeros((_L,), jnp.float32)
        acc2 = jnp.zeros((_L,), jnp.float32)
        for k in range(_KC):
            for j in range(_VPC):
                sl = pl.ds(j * _L, _L)
                d = dat_v[0, k, sl] + dat_v[1, k, sl] - dat_v[2, k, sl]
                acc1 = acc1 + d
                acc2 = acc2 + d * d
        st_v[0, :] = acc2 * (0.5 / _B)
        st_v[1, :] = acc1 * (0.5 / _B)
        pltpu.sync_copy(st_v.at[0], out_hbm.at[pl.ds(wid * _L, _L)])
        pltpu.sync_copy(st_v.at[1], out_hbm.at[pl.ds((_NW + wid) * _L, _L)])

    return _k


_sc_kernel = _make_sc_kernel()


def kernel(sampleU, sampleI, sampleR, alpha, betaU, betaI):
    su = sampleU.astype(jnp.int32).reshape(_NW, _KC, _CW)
    si = sampleI.astype(jnp.int32).reshape(_NW, _KC, _CW)
    r = sampleR.astype(jnp.float32).reshape(_NW, _KC, _CW)
    partials = _sc_kernel(su, si, r, betaU, betaI)
    s = jnp.sum(partials.reshape(2, _NW * _L), axis=1)
    a = alpha.astype(jnp.float32)
    return s[0] + 2.0 * a * s[1] + 0.5 * a * a


# speedup vs baseline: 1.0774x; 1.0774x over previous
import functools

import jax
import jax.numpy as jnp
from jax import lax
from jax.experimental import pallas as pl
from jax.experimental.pallas import tpu as pltpu
from jax.experimental.pallas import tpu_sc as plsc

_NC = 2
_NS = 16
_NW = _NC * _NS
_B = 16384
_BPW = _B // _NW
_CW = 128
_KC = _BPW // _CW
_L = 16
_VPC = _CW // _L


def _make_sc_kernel():
    mesh = plsc.VectorSubcoreMesh(core_axis_name="c", subcore_axis_name="s")

    @functools.partial(
        pl.kernel,
        mesh=mesh,
        out_type=jax.ShapeDtypeStruct((_NW * _L,), jnp.float32),
        scratch_types=[
            pltpu.VMEM((2, _KC, _CW), jnp.int32),
            pltpu.VMEM((3, _KC, _CW), jnp.float32),
            pltpu.VMEM((2, _L), jnp.float32),
            pltpu.SemaphoreType.DMA,
            pltpu.SemaphoreType.DMA,
        ],
    )
    def _k(su_hbm, si_hbm, r_hbm, alpha_hbm, bu_hbm, bi_hbm, out_hbm,
           idx_v, dat_v, aa_v, sem_in, sem_g):
        cid = lax.axis_index("c")
        sid = lax.axis_index("s")
        wid = sid * _NC + cid

        ld_u = pltpu.async_copy(su_hbm.at[wid], idx_v.at[0], sem_in)
        ld_i = pltpu.async_copy(si_hbm.at[wid], idx_v.at[1], sem_in)
        ld_r = pltpu.async_copy(r_hbm.at[wid], dat_v.at[2], sem_in)
        ld_a = pltpu.async_copy(alpha_hbm, aa_v.at[0], sem_in)
        ld_u.wait()
        ld_i.wait()
        copies = []
        for k in range(_KC):
            copies.append(pltpu.async_copy(
                bu_hbm.at[idx_v.at[0, k]], dat_v.at[0, k], sem_g))
            copies.append(pltpu.async_copy(
                bi_hbm.at[idx_v.at[1, k]], dat_v.at[1, k], sem_g))
        ld_r.wait()
        ld_a.wait()
        for c in copies:
            c.wait()

        av = aa_v[0, :]
        acc = jnp.zeros((_L,), jnp.float32)
        for k in range(_KC):
            for j in range(_VPC):
                sl = pl.ds(j * _L, _L)
                d = av + dat_v[0, k, sl] + dat_v[1, k, sl] - dat_v[2, k, sl]
                acc = acc + d * d
        aa_v[1, :] = acc * (0.5 / _B)
        pltpu.sync_copy(aa_v.at[1], out_hbm.at[pl.ds(wid * _L, _L)])

    return _k


_sc_kernel = _make_sc_kernel()


def kernel(sampleU, sampleI, sampleR, alpha, betaU, betaI):
    su = sampleU.astype(jnp.int32).reshape(_NW, _KC, _CW)
    si = sampleI.astype(jnp.int32).reshape(_NW, _KC, _CW)
    r = sampleR.astype(jnp.float32).reshape(_NW, _KC, _CW)
    al = jnp.broadcast_to(alpha.astype(jnp.float32), (_L,))
    partials = _sc_kernel(su, si, r, al, betaU, betaI)
    return jnp.sum(partials)
